# baseline (speedup 1.0000x reference)
import jax
import jax.numpy as jnp
from jax.experimental import pallas as pl
from jax.experimental.pallas import tpu as pltpu

_M_BLK = 200


def _k1(nsadj_ref, nfadj_ref, x_ref, w1a_ref, b1a_ref, w2a_ref,
        w1b_ref, b1b_ref, w2b_ref, ta_ref, tb_ref, sa_ref, sb_ref):
    i = pl.program_id(0)

    @pl.when(i == 0)
    def _():
        sa_ref[...] = jnp.dot(x_ref[...], w1a_ref[...],
                              preferred_element_type=jnp.float32)
        sb_ref[...] = jnp.dot(x_ref[...], w1b_ref[...],
                              preferred_element_type=jnp.float32)

    ha = jnp.maximum(
        jnp.dot(nsadj_ref[...], sa_ref[...],
                preferred_element_type=jnp.float32) + b1a_ref[...], 0.0)
    ta_ref[...] = jnp.dot(ha, w2a_ref[...], preferred_element_type=jnp.float32)
    hb = jnp.maximum(
        jnp.dot(nfadj_ref[...], sb_ref[...],
                preferred_element_type=jnp.float32) + b1b_ref[...], 0.0)
    tb_ref[...] = jnp.dot(hb, w2b_ref[...], preferred_element_type=jnp.float32)


def _k2(nsadj_ref, nfadj_ref, ta_ref, tb_ref, b2a_ref, b2b_ref, wp_ref,
        x1_ref, x2_ref, p1_ref, p2_ref):
    x1 = jnp.dot(nsadj_ref[...], ta_ref[...],
                 preferred_element_type=jnp.float32) + b2a_ref[...]
    x1_ref[...] = x1
    p1_ref[...] = jnp.maximum(
        jnp.dot(jnp.maximum(x1, 0.0), wp_ref[...],
                preferred_element_type=jnp.float32), 0.0)
    x2 = jnp.dot(nfadj_ref[...], tb_ref[...],
                 preferred_element_type=jnp.float32) + b2b_ref[...]
    x2_ref[...] = x2
    p2_ref[...] = jnp.maximum(
        jnp.dot(jnp.maximum(x2, 0.0), wp_ref[...],
                preferred_element_type=jnp.float32), 0.0)


def kernel(X, nsadj, nfadj, W1a, b1a, W2a, b2a, W1b, b1b, W2b, b2b, Wp):
    n, nfeat = X.shape
    nh1 = W1a.shape[1]
    nh2 = W2a.shape[1]
    ncls = Wp.shape[1]
    grid = (n // _M_BLK,)

    _full = lambda shape: pl.BlockSpec(shape, lambda i: (0, 0))
    _rows = lambda w: pl.BlockSpec((_M_BLK, w), lambda i: (i, 0))

    ta, tb = pl.pallas_call(
        _k1,
        grid=grid,
        in_specs=[
            _rows(n), _rows(n),
            _full((n, nfeat)),
            _full((nfeat, nh1)), _full((1, nh1)), _full((nh1, nh2)),
            _full((nfeat, nh1)), _full((1, nh1)), _full((nh1, nh2)),
        ],
        out_specs=[_rows(nh2), _rows(nh2)],
        out_shape=[
            jax.ShapeDtypeStruct((n, nh2), jnp.float32),
            jax.ShapeDtypeStruct((n, nh2), jnp.float32),
        ],
        scratch_shapes=[
            pltpu.VMEM((n, nh1), jnp.float32),
            pltpu.VMEM((n, nh1), jnp.float32),
        ],
        compiler_params=pltpu.CompilerParams(
            dimension_semantics=("arbitrary",),
        ),
    )(nsadj, nfadj, X, W1a, b1a.reshape(1, -1), W2a,
      W1b, b1b.reshape(1, -1), W2b)

    x1, x2, p1, p2 = pl.pallas_call(
        _k2,
        grid=grid,
        in_specs=[
            _rows(n), _rows(n),
            _full((n, nh2)), _full((n, nh2)),
            _full((1, nh2)), _full((1, nh2)),
            _full((nh2, ncls)),
        ],
        out_specs=[_rows(nh2), _rows(nh2), _rows(ncls), _rows(ncls)],
        out_shape=[
            jax.ShapeDtypeStruct((n, nh2), jnp.float32),
            jax.ShapeDtypeStruct((n, nh2), jnp.float32),
            jax.ShapeDtypeStruct((n, ncls), jnp.float32),
            jax.ShapeDtypeStruct((n, ncls), jnp.float32),
        ],
        compiler_params=pltpu.CompilerParams(
            dimension_semantics=("arbitrary",),
        ),
    )(nsadj, nfadj, ta, tb, b2a.reshape(1, -1), b2b.reshape(1, -1), Wp)

    return (p1, p2, x1, x2)


# speedup vs baseline: 1.0938x; 1.0938x over previous
import jax
import jax.numpy as jnp
from jax.experimental import pallas as pl
from jax.experimental.pallas import tpu as pltpu

_M_BLK = 200


def _k1(nsadj_ref, nfadj_ref, x_ref, w1a_ref, b1a_ref, w2a_ref,
        w1b_ref, b1b_ref, w2b_ref, ta_ref, tb_ref, sa_ref, sb_ref):
    i = pl.program_id(0)

    @pl.when(i == 0)
    def _():
        sa_ref[...] = jnp.dot(x_ref[...], w1a_ref[...],
                              preferred_element_type=jnp.float32, precision=jax.lax.Precision.DEFAULT)
        sb_ref[...] = jnp.dot(x_ref[...], w1b_ref[...],
                              preferred_element_type=jnp.float32, precision=jax.lax.Precision.DEFAULT)

    ha = jnp.maximum(
        jnp.dot(nsadj_ref[...].astype(jnp.bfloat16), sa_ref[...].astype(jnp.bfloat16),
                preferred_element_type=jnp.float32) + b1a_ref[...], 0.0)
    ta_ref[...] = jnp.dot(ha, w2a_ref[...], preferred_element_type=jnp.float32, precision=jax.lax.Precision.DEFAULT)
    hb = jnp.maximum(
        jnp.dot(nfadj_ref[...].astype(jnp.bfloat16), sb_ref[...].astype(jnp.bfloat16),
                preferred_element_type=jnp.float32) + b1b_ref[...], 0.0)
    tb_ref[...] = jnp.dot(hb, w2b_ref[...], preferred_element_type=jnp.float32, precision=jax.lax.Precision.DEFAULT)


def _k2(nsadj_ref, nfadj_ref, ta_ref, tb_ref, b2a_ref, b2b_ref, wp_ref,
        x1_ref, x2_ref, p1_ref, p2_ref):
    x1 = jnp.dot(nsadj_ref[...].astype(jnp.bfloat16), ta_ref[...].astype(jnp.bfloat16),
                 preferred_element_type=jnp.float32) + b2a_ref[...]
    x1_ref[...] = x1
    p1_ref[...] = jnp.maximum(
        jnp.dot(jnp.maximum(x1, 0.0), wp_ref[...],
                preferred_element_type=jnp.float32, precision=jax.lax.Precision.DEFAULT), 0.0)
    x2 = jnp.dot(nfadj_ref[...].astype(jnp.bfloat16), tb_ref[...].astype(jnp.bfloat16),
                 preferred_element_type=jnp.float32) + b2b_ref[...]
    x2_ref[...] = x2
    p2_ref[...] = jnp.maximum(
        jnp.dot(jnp.maximum(x2, 0.0), wp_ref[...],
                preferred_element_type=jnp.float32, precision=jax.lax.Precision.DEFAULT), 0.0)


def kernel(X, nsadj, nfadj, W1a, b1a, W2a, b2a, W1b, b1b, W2b, b2b, Wp):
    n, nfeat = X.shape
    nh1 = W1a.shape[1]
    nh2 = W2a.shape[1]
    ncls = Wp.shape[1]
    grid = (n // _M_BLK,)

    _full = lambda shape: pl.BlockSpec(shape, lambda i: (0, 0))
    _rows = lambda w: pl.BlockSpec((_M_BLK, w), lambda i: (i, 0))

    ta, tb = pl.pallas_call(
        _k1,
        grid=grid,
        in_specs=[
            _rows(n), _rows(n),
            _full((n, nfeat)),
            _full((nfeat, nh1)), _full((1, nh1)), _full((nh1, nh2)),
            _full((nfeat, nh1)), _full((1, nh1)), _full((nh1, nh2)),
        ],
        out_specs=[_rows(nh2), _rows(nh2)],
        out_shape=[
            jax.ShapeDtypeStruct((n, nh2), jnp.float32),
            jax.ShapeDtypeStruct((n, nh2), jnp.float32),
        ],
        scratch_shapes=[
            pltpu.VMEM((n, nh1), jnp.float32),
            pltpu.VMEM((n, nh1), jnp.float32),
        ],
        compiler_params=pltpu.CompilerParams(
            dimension_semantics=("arbitrary",),
        ),
    )(nsadj, nfadj, X, W1a, b1a.reshape(1, -1), W2a,
      W1b, b1b.reshape(1, -1), W2b)

    x1, x2, p1, p2 = pl.pallas_call(
        _k2,
        grid=grid,
        in_specs=[
            _rows(n), _rows(n),
            _full((n, nh2)), _full((n, nh2)),
            _full((1, nh2)), _full((1, nh2)),
            _full((nh2, ncls)),
        ],
        out_specs=[_rows(nh2), _rows(nh2), _rows(ncls), _rows(ncls)],
        out_shape=[
            jax.ShapeDtypeStruct((n, nh2), jnp.float32),
            jax.ShapeDtypeStruct((n, nh2), jnp.float32),
            jax.ShapeDtypeStruct((n, ncls), jnp.float32),
            jax.ShapeDtypeStruct((n, ncls), jnp.float32),
        ],
        compiler_params=pltpu.CompilerParams(
            dimension_semantics=("arbitrary",),
        ),
    )(nsadj, nfadj, ta, tb, b2a.reshape(1, -1), b2b.reshape(1, -1), Wp)

    return (p1, p2, x1, x2)
